# Initial kernel scaffold; baseline (speedup 1.0000x reference)
import jax
import jax.numpy as jnp
from jax.experimental import pallas as pl


def kernel(pc, feats, W_res, b_res, W_0, b_0, W_l0, b_l0, W_s0, b_s0, W_f0, b_f0, W_l1, b_l1, W_s1, b_s1, W_f1, b_f1, W_1, b_1):
    raise NotImplementedError("write your pallas kernel here")


# speedup vs baseline: 4.4379x; 4.4379x over previous
import functools

import jax
import jax.numpy as jnp
from jax import lax
from jax.experimental import pallas as pl
from jax.experimental.pallas import tpu as pltpu
from jax.experimental.pallas import tpu_sc as plsc

_K = 16
_HIGH = lax.Precision.HIGHEST


def _dot(a, b):
    return lax.dot_general(a, b, (((1,), (0,)), ((), ())),
                           precision=_HIGH,
                           preferred_element_type=jnp.float32)


def _knn_body(n_pts, bn, pc_ref, pct_ref, idx_ref):
    b = pl.program_id(0)
    a = pc_ref[0]
    t = pct_ref[0]
    d2 = jnp.zeros((bn, n_pts), jnp.float32)
    for d in range(3):
        diff = a[:, d:d + 1] - t[d:d + 1, :]
        d2 = d2 + diff * diff
    colid = lax.broadcasted_iota(jnp.int32, (bn, n_pts), 1)
    inf = jnp.float32(jnp.inf)
    cols = []
    for _ in range(_K):
        m = jnp.min(d2, axis=1, keepdims=True)
        am = jnp.min(jnp.where(d2 <= m, colid, n_pts), axis=1, keepdims=True)
        cols.append(am)
        d2 = jnp.where(colid == am, inf, d2)
    idx_ref[0] = jnp.concatenate(cols, axis=1) + b * n_pts


def _knn(pc):
    bsz, n_pts, _ = pc.shape
    bn = 256
    pct = jnp.transpose(pc, (0, 2, 1))
    return pl.pallas_call(
        functools.partial(_knn_body, n_pts, bn),
        grid=(bsz, n_pts // bn),
        in_specs=[
            pl.BlockSpec((1, bn, 3), lambda b, i: (b, i, 0)),
            pl.BlockSpec((1, 3, n_pts), lambda b, i: (b, 0, 0)),
        ],
        out_specs=pl.BlockSpec((1, bn, _K), lambda b, i: (b, i, 0)),
        out_shape=jax.ShapeDtypeStruct((bsz, n_pts, _K), jnp.int32),
    )(pc, pct)


def _pre_body(f_ref, wres_ref, bres_ref, w0_ref, b0_ref, wst_ref,
              y_ref, x0_ref, z0_ref):
    f = f_ref[...]
    y_ref[...] = jnp.maximum(_dot(f, wres_ref[...]) + bres_ref[...], 0.0)
    x0 = jnp.maximum(_dot(f, w0_ref[...]) + b0_ref[...], 0.0)
    x0_ref[...] = x0
    z0_ref[...] = _dot(x0, wst_ref[...])


def _pre(f2, w_res, b_res, w_0, b_0, ws0_top):
    m, f_dim = f2.shape
    u = w_res.shape[1]
    u4 = w_0.shape[1]
    u2 = ws0_top.shape[1]
    rb = 512
    full = lambda shp: pl.BlockSpec(shp, lambda i: tuple(0 for _ in shp))
    return pl.pallas_call(
        _pre_body,
        grid=(m // rb,),
        in_specs=[
            pl.BlockSpec((rb, f_dim), lambda i: (i, 0)),
            full((f_dim, u)), full((1, u)),
            full((f_dim, u4)), full((1, u4)),
            full((u4, u2)),
        ],
        out_specs=[
            pl.BlockSpec((rb, u), lambda i: (i, 0)),
            pl.BlockSpec((rb, u4), lambda i: (i, 0)),
            pl.BlockSpec((rb, u2), lambda i: (i, 0)),
        ],
        out_shape=[
            jax.ShapeDtypeStruct((m, u), jnp.float32),
            jax.ShapeDtypeStruct((m, u4), jnp.float32),
            jax.ShapeDtypeStruct((m, u2), jnp.float32),
        ],
    )(f2, w_res, b_res, w_0, b_0, ws0_top)


def _sc_gather_tables(tables, idxg):
    n_t = len(tables)
    r_tot = idxg.shape[0]
    info = plsc.get_sparse_core_info()
    nw = info.num_cores * info.num_subcores
    per_w = r_tot // nw
    chunk = 128
    n_it = per_w // chunk
    mesh = plsc.VectorSubcoreMesh(core_axis_name="c", subcore_axis_name="s")

    out_type = [jax.ShapeDtypeStruct((r_tot, t.shape[1]), jnp.float32)
                for t in tables]
    scratch_types = ([pltpu.VMEM((chunk,), jnp.int32)]
                     + [pltpu.VMEM((chunk, t.shape[1]), jnp.float32)
                        for t in tables]
                     + [pltpu.SemaphoreType.DMA])

    def body(*refs):
        tabs = refs[0:n_t]
        idx_hbm = refs[n_t]
        outs = refs[n_t + 1:2 * n_t + 1]
        idx_v = refs[2 * n_t + 1]
        bufs = refs[2 * n_t + 2:3 * n_t + 2]
        sem = refs[3 * n_t + 2]
        wid = lax.axis_index("s") * info.num_cores + lax.axis_index("c")

        def step(j, carry):
            base = wid * per_w + j * chunk
            pltpu.sync_copy(idx_hbm.at[pl.ds(base, chunk)], idx_v)
            for tb, bf in zip(tabs, bufs):
                pltpu.async_copy(tb.at[idx_v], bf, sem).wait()
            for bf, ob in zip(bufs, outs):
                pltpu.sync_copy(bf, ob.at[pl.ds(base, chunk)])
            return carry

        lax.fori_loop(0, n_it, step, 0)

    fn = pl.kernel(body, mesh=mesh, out_type=out_type,
                   scratch_types=scratch_types)
    res = fn(*tables, idxg)
    return list(res) if isinstance(res, (tuple, list)) else [res]


def _att_core(pb, u4, xg, zg, pg, pr, wa, wb, wn, bl, wsb, bs):
    pg = pg[:, :16]
    diff = pr - pg
    norms = jnp.sqrt(jnp.sum(diff * diff, axis=1, keepdims=True))
    pre = _dot(pr, wa) + _dot(pg, wb) + norms * wn + bl
    r = jnp.maximum(pre, 0.0)
    logits = zg + _dot(r, wsb) + bs
    mx = jnp.max(logits, axis=1, keepdims=True)
    e = jnp.exp(logits - mx)
    s = e / jnp.sum(e, axis=1, keepdims=True)
    wnf = xg * s[:, :u4]
    wr = r * s[:, u4:]
    anf = jnp.sum(wnf.reshape(pb, _K, u4), axis=1)
    ar = jnp.sum(wr.reshape(pb, _K, u4), axis=1)
    return anf, ar


def _layer0_body(pb, u4,
                 xg_ref, zg_ref, pg_ref, pr_ref, wa_ref, wb_ref, wn_ref,
                 bl_ref, wsb_ref, bs_ref, wft_ref, wfb_ref, bf_ref, wst_ref,
                 x1_ref, z1_ref):
    anf, ar = _att_core(pb, u4, xg_ref[...], zg_ref[...], pg_ref[...],
                        pr_ref[...], wa_ref[...], wb_ref[...], wn_ref[...],
                        bl_ref[...], wsb_ref[...], bs_ref[...])
    x1 = jnp.maximum(_dot(anf, wft_ref[...]) + _dot(ar, wfb_ref[...])
                     + bf_ref[...], 0.0)
    x1_ref[...] = x1
    z1_ref[...] = _dot(x1, wst_ref[...])


def _layer1_body(pb, u4,
                 xg_ref, zg_ref, pg_ref, pr_ref, wa_ref, wb_ref, wn_ref,
                 bl_ref, wsb_ref, bs_ref, wft_ref, wfb_ref, bf_ref,
                 w1_ref, b1_ref, y_ref, out_ref):
    anf, ar = _att_core(pb, u4, xg_ref[...], zg_ref[...], pg_ref[...],
                        pr_ref[...], wa_ref[...], wb_ref[...], wn_ref[...],
                        bl_ref[...], wsb_ref[...], bs_ref[...])
    x2 = jnp.maximum(_dot(anf, wft_ref[...]) + _dot(ar, wfb_ref[...])
                     + bf_ref[...], 0.0)
    h = jnp.maximum(_dot(x2, w1_ref[...]) + b1_ref[...], 0.0)
    o = h + y_ref[...]
    out_ref[...] = jnp.where(o >= 0.0, o, 0.2 * o)


def _row_spec(rows, cols):
    return pl.BlockSpec((rows, cols), lambda i: (i, 0))


def _full_spec(shp):
    return pl.BlockSpec(shp, lambda i: tuple(0 for _ in shp))


def _layer0(xg, zg, pg, pr, wa, wb, wn, bl, wsb, bs, wft, wfb, bf, wst):
    r_tot, u4 = xg.shape
    u2 = zg.shape[1]
    rb = 1024
    pb = rb // _K
    m = r_tot // _K
    weights = [wa, wb, wn, bl, wsb, bs, wft, wfb, bf, wst]
    return pl.pallas_call(
        functools.partial(_layer0_body, pb, u4),
        grid=(r_tot // rb,),
        in_specs=[
            _row_spec(rb, u4), _row_spec(rb, u2),
            _row_spec(rb, 128), _row_spec(rb, 16),
        ] + [_full_spec(w.shape) for w in weights],
        out_specs=[_row_spec(pb, u4), _row_spec(pb, u2)],
        out_shape=[
            jax.ShapeDtypeStruct((m, u4), jnp.float32),
            jax.ShapeDtypeStruct((m, u2), jnp.float32),
        ],
    )(xg, zg, pg, pr, *weights)


def _layer1(xg, zg, pg, pr, wa, wb, wn, bl, wsb, bs, wft, wfb, bf,
            w1, b1, y):
    r_tot, u4 = xg.shape
    u2 = zg.shape[1]
    u = w1.shape[1]
    rb = 1024
    pb = rb // _K
    m = r_tot // _K
    weights = [wa, wb, wn, bl, wsb, bs, wft, wfb, bf, w1, b1]
    return pl.pallas_call(
        functools.partial(_layer1_body, pb, u4),
        grid=(r_tot // rb,),
        in_specs=[
            _row_spec(rb, u4), _row_spec(rb, u2),
            _row_spec(rb, 128), _row_spec(rb, 16),
        ] + [_full_spec(w.shape) for w in weights]
          + [_row_spec(pb, u)],
        out_specs=_row_spec(pb, u),
        out_shape=jax.ShapeDtypeStruct((m, u), jnp.float32),
    )(xg, zg, pg, pr, *weights, y)


def _fold_locse_weights(wl):
    wa = jnp.pad(wl[0:3] + wl[6:9], ((0, 13), (0, 0)))
    wb = jnp.pad(wl[3:6] - wl[6:9], ((0, 13), (0, 0)))
    return wa, wb, wl[9:10]


def kernel(pc, feats, W_res, b_res, W_0, b_0, W_l0, b_l0, W_s0, b_s0,
           W_f0, b_f0, W_l1, b_l1, W_s1, b_s1, W_f1, b_f1, W_1, b_1):
    bsz, n_pts, dims = pc.shape
    m = bsz * n_pts
    f_dim = feats.shape[-1]
    u4 = W_0.shape[1]
    u2 = 2 * u4
    u = W_res.shape[1]

    idx = _knn(pc)
    idxg = idx.reshape(m * _K)

    y, x0, z0 = _pre(feats.reshape(m, f_dim), W_res, b_res.reshape(1, u),
                     W_0, b_0.reshape(1, u4), W_s0[:u4])

    pcp = jnp.pad(pc.reshape(m, dims), ((0, 0), (0, 128 - dims)))
    pcr = jnp.broadcast_to(pcp[:, :16].reshape(m, 1, 16),
                           (m, _K, 16)).reshape(m * _K, 16)

    wa0, wb0, wn0 = _fold_locse_weights(W_l0)
    wa1, wb1, wn1 = _fold_locse_weights(W_l1)

    xg0, zg0, pg = _sc_gather_tables([x0, z0, pcp], idxg)
    x1, z1 = _layer0(xg0, zg0, pg, pcr,
                     wa0, wb0, wn0, b_l0.reshape(1, u4),
                     W_s0[u4:], b_s0.reshape(1, u2),
                     W_f0[:u4], W_f0[u4:], b_f0.reshape(1, u4),
                     W_s1[:u4])

    xg1, zg1 = _sc_gather_tables([x1, z1], idxg)
    out = _layer1(xg1, zg1, pg, pcr,
                  wa1, wb1, wn1, b_l1.reshape(1, u4),
                  W_s1[u4:], b_s1.reshape(1, u2),
                  W_f1[:u4], W_f1[u4:], b_f1.reshape(1, u2),
                  W_1, b_1.reshape(1, u), y)
    return out.reshape(bsz, n_pts, u)


# speedup vs baseline: 11.1916x; 2.5218x over previous
import functools

import jax
import jax.numpy as jnp
from jax import lax
from jax.experimental import pallas as pl
from jax.experimental.pallas import tpu as pltpu
from jax.experimental.pallas import tpu_sc as plsc

_K = 16


def _dot(a, b):
    return lax.dot_general(a, b, (((1,), (0,)), ((), ())),
                           preferred_element_type=jnp.float32)


def _knn_body(n_pts, bn, pc_ref, pct_ref, idx_ref):
    b = pl.program_id(0)
    a = pc_ref[0]
    t = pct_ref[0]
    d2 = jnp.zeros((bn, n_pts), jnp.float32)
    for d in range(3):
        diff = a[:, d:d + 1] - t[d:d + 1, :]
        d2 = d2 + diff * diff
    colid = lax.broadcasted_iota(jnp.int32, (bn, n_pts), 1)
    key = (lax.bitcast_convert_type(d2, jnp.int32) & ~2047) | colid
    big = jnp.int32(jnp.iinfo(jnp.int32).max)
    cols = []
    for _ in range(_K):
        m = jnp.min(key, axis=1, keepdims=True)
        cols.append(m & 2047)
        key = jnp.where(key == m, big, key)
    idx_ref[0] = jnp.concatenate(cols, axis=1) + b * n_pts


def _knn(pc):
    bsz, n_pts, dims = pc.shape
    bn = 256
    pc8 = jnp.pad(pc, ((0, 0), (0, 0), (0, 8 - dims)))
    pct = jnp.transpose(pc8, (0, 2, 1))
    return pl.pallas_call(
        functools.partial(_knn_body, n_pts, bn),
        grid=(bsz, n_pts // bn),
        in_specs=[
            pl.BlockSpec((1, bn, 8), lambda b, i: (b, i, 0)),
            pl.BlockSpec((1, 8, n_pts), lambda b, i: (b, 0, 0)),
        ],
        out_specs=pl.BlockSpec((1, bn, _K), lambda b, i: (b, i, 0)),
        out_shape=jax.ShapeDtypeStruct((bsz, n_pts, _K), jnp.int32),
    )(pc8, pct)


def _pre_body(f_ref, wres_ref, bres_ref, w0_ref, b0_ref, y_ref, x0_ref):
    f = f_ref[...]
    y_ref[...] = jnp.maximum(_dot(f, wres_ref[...]) + bres_ref[...], 0.0)
    x0_ref[...] = jnp.maximum(_dot(f, w0_ref[...]) + b0_ref[...], 0.0)


def _pre(f2, w_res, b_res, w_0, b_0):
    m, f_dim = f2.shape
    u = w_res.shape[1]
    u4 = w_0.shape[1]
    rb = 512
    full = lambda shp: pl.BlockSpec(shp, lambda i: tuple(0 for _ in shp))
    return pl.pallas_call(
        _pre_body,
        grid=(m // rb,),
        in_specs=[
            pl.BlockSpec((rb, f_dim), lambda i: (i, 0)),
            full((f_dim, u)), full((1, u)),
            full((f_dim, u4)), full((1, u4)),
        ],
        out_specs=[
            pl.BlockSpec((rb, u), lambda i: (i, 0)),
            pl.BlockSpec((rb, u4), lambda i: (i, 0)),
        ],
        out_shape=[
            jax.ShapeDtypeStruct((m, u), jnp.float32),
            jax.ShapeDtypeStruct((m, u4), jnp.float32),
        ],
    )(f2, w_res, b_res, w_0, b_0)


def _sc_gather_tables(tables, idxg):
    n_t = len(tables)
    r_tot = idxg.shape[0]
    info = plsc.get_sparse_core_info()
    nw = info.num_cores * info.num_subcores
    per_w = r_tot // nw
    chunk = 128
    n_it = per_w // chunk
    mesh = plsc.VectorSubcoreMesh(core_axis_name="c", subcore_axis_name="s")

    out_type = [jax.ShapeDtypeStruct((r_tot, t.shape[1]), jnp.float32)
                for t in tables]
    scratch_types = ([pltpu.VMEM((chunk,), jnp.int32)] * 2
                     + [pltpu.VMEM((chunk, t.shape[1]), jnp.float32)
                        for t in tables] * 2
                     + [pltpu.SemaphoreType.DMA] * 2)

    def body(*refs):
        tabs = refs[0:n_t]
        idx_hbm = refs[n_t]
        outs = refs[n_t + 1:2 * n_t + 1]
        scr = refs[2 * n_t + 1:]
        idx_vs = scr[0:2]
        bufs = [scr[2 + n_t * h:2 + n_t * (h + 1)] for h in range(2)]
        sems = scr[2 + 2 * n_t:2 + 2 * n_t + 2]
        wid = lax.axis_index("s") * info.num_cores + lax.axis_index("c")

        def step(jj, carry):
            handles = []
            for h in range(2):
                base = wid * per_w + (2 * jj + h) * chunk
                pltpu.sync_copy(idx_hbm.at[pl.ds(base, chunk)], idx_vs[h])
                handles.append([
                    pltpu.async_copy(tb.at[idx_vs[h]], bf, sems[h])
                    for tb, bf in zip(tabs, bufs[h])])
            for h in range(2):
                for hd in handles[h]:
                    hd.wait()
                base = wid * per_w + (2 * jj + h) * chunk
                for bf, ob in zip(bufs[h], outs):
                    pltpu.sync_copy(bf, ob.at[pl.ds(base, chunk)])
            return carry

        lax.fori_loop(0, n_it // 2, step, 0)

    fn = pl.kernel(body, mesh=mesh, out_type=out_type,
                   scratch_types=scratch_types)
    res = fn(*tables, idxg)
    return list(res) if isinstance(res, (tuple, list)) else [res]


def _att_core(pb, u4, xg, pg, pr, wa, wb, wn, bl, ws, bs):
    pg = pg[:, :16]
    diff = pr - pg
    norms = jnp.sqrt(jnp.sum(diff * diff, axis=1, keepdims=True))
    pre = _dot(pr, wa) + _dot(pg, wb) + norms * wn + bl
    r = jnp.maximum(pre, 0.0)
    nf = jnp.concatenate([xg, r], axis=1)
    logits = _dot(nf, ws) + bs
    mx = jnp.max(logits, axis=1, keepdims=True)
    e = jnp.exp(logits - mx)
    s = e / jnp.sum(e, axis=1, keepdims=True)
    w = nf * s
    att = jnp.sum(w.reshape(pb, _K, 2 * u4), axis=1)
    return att


def _layer0_body(pb, u4,
                 xg_ref, pg_ref, pr_ref, wa_ref, wb_ref, wn_ref,
                 bl_ref, ws_ref, bs_ref, wf_ref, bf_ref, x1_ref):
    att = _att_core(pb, u4, xg_ref[...], pg_ref[...], pr_ref[...],
                    wa_ref[...], wb_ref[...], wn_ref[...],
                    bl_ref[...], ws_ref[...], bs_ref[...])
    x1_ref[...] = jnp.maximum(_dot(att, wf_ref[...]) + bf_ref[...], 0.0)


def _layer1_body(pb, u4,
                 xg_ref, pg_ref, pr_ref, wa_ref, wb_ref, wn_ref,
                 bl_ref, ws_ref, bs_ref, wf_ref, bf_ref,
                 w1_ref, b1_ref, y_ref, out_ref):
    att = _att_core(pb, u4, xg_ref[...], pg_ref[...], pr_ref[...],
                    wa_ref[...], wb_ref[...], wn_ref[...],
                    bl_ref[...], ws_ref[...], bs_ref[...])
    x2 = jnp.maximum(_dot(att, wf_ref[...]) + bf_ref[...], 0.0)
    h = jnp.maximum(_dot(x2, w1_ref[...]) + b1_ref[...], 0.0)
    o = h + y_ref[...]
    out_ref[...] = jnp.where(o >= 0.0, o, 0.2 * o)


def _row_spec(rows, cols):
    return pl.BlockSpec((rows, cols), lambda i: (i, 0))


def _full_spec(shp):
    return pl.BlockSpec(shp, lambda i: tuple(0 for _ in shp))


def _layer0(xg, pg, pr, wa, wb, wn, bl, ws, bs, wf, bf):
    r_tot, u4 = xg.shape
    rb = 1024
    pb = rb // _K
    m = r_tot // _K
    weights = [wa, wb, wn, bl, ws, bs, wf, bf]
    return pl.pallas_call(
        functools.partial(_layer0_body, pb, u4),
        grid=(r_tot // rb,),
        in_specs=[
            _row_spec(rb, u4), _row_spec(rb, 128), _row_spec(rb, 16),
        ] + [_full_spec(w.shape) for w in weights],
        out_specs=_row_spec(pb, u4),
        out_shape=jax.ShapeDtypeStruct((m, u4), jnp.float32),
    )(xg, pg, pr, *weights)


def _layer1(xg, pg, pr, wa, wb, wn, bl, ws, bs, wf, bf, w1, b1, y):
    r_tot, u4 = xg.shape
    u = w1.shape[1]
    rb = 1024
    pb = rb // _K
    m = r_tot // _K
    weights = [wa, wb, wn, bl, ws, bs, wf, bf, w1, b1]
    return pl.pallas_call(
        functools.partial(_layer1_body, pb, u4),
        grid=(r_tot // rb,),
        in_specs=[
            _row_spec(rb, u4), _row_spec(rb, 128), _row_spec(rb, 16),
        ] + [_full_spec(w.shape) for w in weights]
          + [_row_spec(pb, u)],
        out_specs=_row_spec(pb, u),
        out_shape=jax.ShapeDtypeStruct((m, u), jnp.float32),
    )(xg, pg, pr, *weights, y)


def _fold_locse_weights(wl):
    wa = jnp.pad(wl[0:3] + wl[6:9], ((0, 13), (0, 0)))
    wb = jnp.pad(wl[3:6] - wl[6:9], ((0, 13), (0, 0)))
    return wa, wb, wl[9:10]


def kernel(pc, feats, W_res, b_res, W_0, b_0, W_l0, b_l0, W_s0, b_s0,
           W_f0, b_f0, W_l1, b_l1, W_s1, b_s1, W_f1, b_f1, W_1, b_1):
    bsz, n_pts, dims = pc.shape
    m = bsz * n_pts
    f_dim = feats.shape[-1]
    u4 = W_0.shape[1]
    u2 = 2 * u4
    u = W_res.shape[1]

    idx = _knn(pc)
    idxg = idx.reshape(m * _K)

    y, x0 = _pre(feats.reshape(m, f_dim), W_res, b_res.reshape(1, u),
                 W_0, b_0.reshape(1, u4))

    pcp = jnp.pad(pc.reshape(m, dims), ((0, 0), (0, 128 - dims)))
    pcr = jnp.broadcast_to(pcp[:, :16].reshape(m, 1, 16),
                           (m, _K, 16)).reshape(m * _K, 16)

    wa0, wb0, wn0 = _fold_locse_weights(W_l0)
    wa1, wb1, wn1 = _fold_locse_weights(W_l1)

    xg0, pg = _sc_gather_tables([x0, pcp], idxg)
    x1 = _layer0(xg0, pg, pcr,
                 wa0, wb0, wn0, b_l0.reshape(1, u4),
                 W_s0, b_s0.reshape(1, u2),
                 W_f0, b_f0.reshape(1, u4))

    (xg1,) = _sc_gather_tables([x1], idxg)
    out = _layer1(xg1, pg, pcr,
                  wa1, wb1, wn1, b_l1.reshape(1, u4),
                  W_s1, b_s1.reshape(1, u2),
                  W_f1, b_f1.reshape(1, u2),
                  W_1, b_1.reshape(1, u), y)
    return out.reshape(bsz, n_pts, u)
